# baseline (speedup 1.0000x reference)
import functools

import jax
import jax.numpy as jnp
from jax import lax
from jax.experimental import pallas as pl
from jax.experimental.pallas import tpu as pltpu
from jax.experimental.pallas import tpu_sc as plsc

_N = 100000
_D = 128
_B = 16384
_MOM = 0.9

_NC = 2
_NS = 16
_NW = _NC * _NS
_BPW = _B // _NW
_ROWS_PW = 3128
_MAP_VREGS = (_ROWS_PW + 15) // 16
_MAP_PAD = _MAP_VREGS * 16
_CAP = ((_ROWS_PW + 127) // 128 + 1) * 128
_CHUNK = 128

_mesh = plsc.VectorSubcoreMesh(
    core_axis_name="c", subcore_axis_name="s", num_cores=_NC, num_subcores=_NS
)


def _wid():
    return lax.axis_index("s") * _NC + lax.axis_index("c")


def _lane_gather(x, i):
    dnums = lax.GatherDimensionNumbers(
        offset_dims=(), collapsed_slice_dims=(0,), start_index_map=(0,))
    return lax.gather(x, i[:, None], dnums, (1,),
                      mode=lax.GatherScatterMode.PROMISE_IN_BOUNDS)


@functools.partial(
    pl.kernel,
    out_type=(
        jax.ShapeDtypeStruct((_B, _D), jnp.float32),
        jax.ShapeDtypeStruct((_B, _D), jnp.float32),
    ),
    mesh=_mesh,
    scratch_types=[
        pltpu.VMEM((_BPW,), jnp.int32),
        pltpu.VMEM((_CHUNK, _D), jnp.float32),
        pltpu.VMEM((_CHUNK, _D), jnp.float32),
        pltpu.SemaphoreType.DMA,
        pltpu.SemaphoreType.DMA,
    ],
)
def _sc_gather(idx_hbm, hidden_hbm, variance_hbm, hold_hbm, varold_hbm,
               idx_v, rowh_v, rowv_v, sem1, sem2):
    base = _wid() * _BPW
    pltpu.sync_copy(idx_hbm.at[pl.ds(base, _BPW)], idx_v)
    for ch in range(_BPW // _CHUNK):
        sl = idx_v.at[pl.ds(ch * _CHUNK, _CHUNK)]
        ch1 = pltpu.async_copy(hidden_hbm.at[sl], rowh_v, sem1)
        ch2 = pltpu.async_copy(variance_hbm.at[sl], rowv_v, sem2)
        ch1.wait()
        pltpu.sync_copy(rowh_v, hold_hbm.at[pl.ds(base + ch * _CHUNK, _CHUNK)])
        ch2.wait()
        pltpu.sync_copy(rowv_v, varold_hbm.at[pl.ds(base + ch * _CHUNK, _CHUNK)])


_BM = 1024


def _gru_body(x_ref, h_ref, v_ref, wih_ref, whh_ref, bih_ref, bhh_ref,
              hn_ref, vn_ref):
    x = x_ref[...]
    h = h_ref[...]
    gi = jnp.dot(x, wih_ref[...], preferred_element_type=jnp.float32) + bih_ref[...]
    gh = jnp.dot(h, whh_ref[...], preferred_element_type=jnp.float32) + bhh_ref[...]
    r = jax.nn.sigmoid(gi[:, :_D] + gh[:, :_D])
    z = jax.nn.sigmoid(gi[:, _D:2 * _D] + gh[:, _D:2 * _D])
    n = jnp.tanh(gi[:, 2 * _D:] + r * gh[:, 2 * _D:])
    hn = (1.0 - z) * n + z * h
    hn_ref[...] = hn
    d = hn - h
    vn_ref[...] = _MOM * v_ref[...] + (1.0 - _MOM) * d * d


def _tc_gru(x, h_old, var_old, wih_t, whh_t, b_ih, b_hh):
    grid = (_B // _BM,)
    row_spec = pl.BlockSpec((_BM, _D), lambda i: (i, 0))
    full_w = pl.BlockSpec((_D, 3 * _D), lambda i: (0, 0))
    full_b = pl.BlockSpec((1, 3 * _D), lambda i: (0, 0))
    return pl.pallas_call(
        _gru_body,
        grid=grid,
        in_specs=[row_spec, row_spec, row_spec, full_w, full_w, full_b, full_b],
        out_specs=[row_spec, row_spec],
        out_shape=[
            jax.ShapeDtypeStruct((_B, _D), jnp.float32),
            jax.ShapeDtypeStruct((_B, _D), jnp.float32),
        ],
    )(x, h_old, var_old, wih_t, whh_t, b_ih, b_hh)


@functools.partial(
    pl.kernel,
    out_type=jax.ShapeDtypeStruct((2 * _N, _D), jnp.float32),
    mesh=_mesh,
    scratch_types=[
        pltpu.VMEM((_B,), jnp.int32),
        pltpu.VMEM((_MAP_PAD,), jnp.int32),
        pltpu.VMEM((_CAP,), jnp.int32),
        pltpu.VMEM((_CAP,), jnp.int32),
        pltpu.VMEM((_CAP,), jnp.int32),
        pltpu.VMEM((16, _D), jnp.float32),
        pltpu.VMEM((16, _D), jnp.float32),
        pltpu.SemaphoreType.DMA,
        pltpu.SemaphoreType.DMA,
        pltpu.SemaphoreType.DMA,
        pltpu.SemaphoreType.DMA,
    ],
    compiler_params=pltpu.CompilerParams(needs_layout_passes=False),
)
def _sc_scatter(idx_hbm, hnew_hbm, varnew_hbm, hidden_hbm, variance_hbm,
                out_hbm, idx_v, win_v, dest_c, destn_c, src_c,
                rowh_v, rowv_v, semc1, semc2, sem1, sem2):
    wid = lax.axis_index("c") * _NS + lax.axis_index("s")
    lo = wid * _ROWS_PW
    hi = jnp.minimum(lo + _ROWS_PW, _N)
    lo_cp = pl.multiple_of(jnp.minimum(lo, _N - _ROWS_PW), 8)
    lo_cp_v = pl.multiple_of(_N + lo_cp, 8)

    cp_h = pltpu.async_copy(
        hidden_hbm.at[pl.ds(lo_cp, _ROWS_PW)],
        out_hbm.at[pl.ds(lo_cp, _ROWS_PW)], semc1)
    cp_v = pltpu.async_copy(
        variance_hbm.at[pl.ds(lo_cp, _ROWS_PW)],
        out_hbm.at[pl.ds(lo_cp_v, _ROWS_PW)], semc2)

    pltpu.sync_copy(idx_hbm, idx_v)
    iot = lax.iota(jnp.int32, 16)

    def init_body(i, _):
        win_v[pl.ds(i * 16, 16)] = jnp.full((16,), -1, jnp.int32)
        return 0

    lax.fori_loop(0, _MAP_VREGS, init_body, 0)

    def scan_body(c, _):
        chunk = idx_v[pl.ds(c * 16, 16)]
        k2 = chunk * 16 + iot
        pos = c * 16 + iot
        k2s, poss = plsc.sort_key_val(k2, pos)
        idxs = lax.shift_right_arithmetic(k2s, 4)
        nxt = _lane_gather(idxs, jnp.minimum(iot + 1, 15))
        bound = (idxs != nxt) | (iot == 15)
        inr = (idxs >= lo) & (idxs < hi)
        plsc.store_scatter(win_v, [idxs - lo], poss, mask=bound & inr)
        return 0

    lax.fori_loop(0, _B // 16, scan_body, 0)

    def compact_body(i, off):
        v = win_v[pl.ds(i * 16, 16)]
        m = v >= 0
        dvals = lo + i * 16 + iot
        plsc.store_compressed(dest_c.at[pl.ds(off, 16)], dvals, mask=m)
        plsc.store_compressed(destn_c.at[pl.ds(off, 16)], dvals + _N, mask=m)
        plsc.store_compressed(src_c.at[pl.ds(off, 16)], v, mask=m)
        return off + jnp.sum(m.astype(jnp.int32))

    k_cnt = lax.fori_loop(0, _MAP_VREGS, compact_body, 0)

    @pl.when(k_cnt > 0)
    def _pad():
        zz = jnp.zeros((16,), jnp.int32)
        dest_c[pl.ds(k_cnt, 16)] = _lane_gather(dest_c[pl.ds(0, 16)], zz)
        destn_c[pl.ds(k_cnt, 16)] = _lane_gather(destn_c[pl.ds(0, 16)], zz)
        src_c[pl.ds(k_cnt, 16)] = _lane_gather(src_c[pl.ds(0, 16)], zz)

    cp_h.wait()
    cp_v.wait()
    plsc.subcore_barrier()

    n_chunks = (k_cnt + 15) // 16

    def chunk_body(j, _):
        d16 = dest_c[pl.ds(j * 16, 16)]
        dn16 = destn_c[pl.ds(j * 16, 16)]
        s16 = src_c[pl.ds(j * 16, 16)]
        g1 = pltpu.async_copy(hnew_hbm.at[s16], rowh_v, sem1)
        g2 = pltpu.async_copy(varnew_hbm.at[s16], rowv_v, sem2)
        g1.wait()
        s1 = pltpu.async_copy(rowh_v, out_hbm.at[d16], sem1)
        g2.wait()
        s2 = pltpu.async_copy(rowv_v, out_hbm.at[dn16], sem2)
        s1.wait()
        s2.wait()
        return 0

    lax.fori_loop(0, n_chunks, chunk_body, 0)


def kernel(x, idx, hidden, variance, W_ih, W_hh, b_ih, b_hh):
    idx = idx.astype(jnp.int32)
    h_old, var_old = _sc_gather(idx, hidden, variance)
    h_new, var_new = _tc_gru(
        x, h_old, var_old, W_ih.T, W_hh.T,
        b_ih.reshape(1, 3 * _D), b_hh.reshape(1, 3 * _D))
    out2 = _sc_scatter(idx, h_new, var_new, hidden, variance)
    return out2.reshape(2, _N, _D)


# speedup vs baseline: 17.6679x; 17.6679x over previous
import functools

import jax
import jax.numpy as jnp
from jax import lax
from jax.experimental import pallas as pl
from jax.experimental.pallas import tpu as pltpu
from jax.experimental.pallas import tpu_sc as plsc
from jax._src.pallas import mpmd as _mpmd

_N = 100000
_D = 128
_B = 16384
_MOM = 0.9

_NC = 2
_NS = 16
_NW = _NC * _NS
_BPW = _B // _NW
_ROWS_PW = 3128
_MAP_VREGS = (_ROWS_PW + 15) // 16
_MAP_PAD = _MAP_VREGS * 16
_CAP = ((_ROWS_PW + 127) // 128 + 1) * 128
_CHUNK = 128

_mesh = plsc.VectorSubcoreMesh(
    core_axis_name="c", subcore_axis_name="s", num_cores=_NC, num_subcores=_NS
)


def _wid():
    return lax.axis_index("s") * _NC + lax.axis_index("c")


def _lane_gather(x, i):
    dnums = lax.GatherDimensionNumbers(
        offset_dims=(), collapsed_slice_dims=(0,), start_index_map=(0,))
    return lax.gather(x, i[:, None], dnums, (1,),
                      mode=lax.GatherScatterMode.PROMISE_IN_BOUNDS)


@functools.partial(
    pl.kernel,
    out_type=(
        jax.ShapeDtypeStruct((_B, _D), jnp.float32),
        jax.ShapeDtypeStruct((_B, _D), jnp.float32),
    ),
    mesh=_mesh,
    scratch_types=[
        pltpu.VMEM((_BPW,), jnp.int32),
        pltpu.VMEM((_CHUNK, _D), jnp.float32),
        pltpu.VMEM((_CHUNK, _D), jnp.float32),
        pltpu.SemaphoreType.DMA,
        pltpu.SemaphoreType.DMA,
    ],
)
def _sc_gather(idx_hbm, hidden_hbm, variance_hbm, hold_hbm, varold_hbm,
               idx_v, rowh_v, rowv_v, sem1, sem2):
    base = _wid() * _BPW
    pltpu.sync_copy(idx_hbm.at[pl.ds(base, _BPW)], idx_v)
    for ch in range(_BPW // _CHUNK):
        sl = idx_v.at[pl.ds(ch * _CHUNK, _CHUNK)]
        ch1 = pltpu.async_copy(hidden_hbm.at[sl], rowh_v, sem1)
        ch2 = pltpu.async_copy(variance_hbm.at[sl], rowv_v, sem2)
        ch1.wait()
        pltpu.sync_copy(rowh_v, hold_hbm.at[pl.ds(base + ch * _CHUNK, _CHUNK)])
        ch2.wait()
        pltpu.sync_copy(rowv_v, varold_hbm.at[pl.ds(base + ch * _CHUNK, _CHUNK)])


_BM = 1024


def _gru_body(x_ref, h_ref, v_ref, wih_ref, whh_ref, bih_ref, bhh_ref,
              hn_ref, vn_ref):
    x = x_ref[...]
    h = h_ref[...]
    gi = jnp.dot(x, wih_ref[...], preferred_element_type=jnp.float32) + bih_ref[...]
    gh = jnp.dot(h, whh_ref[...], preferred_element_type=jnp.float32) + bhh_ref[...]
    r = jax.nn.sigmoid(gi[:, :_D] + gh[:, :_D])
    z = jax.nn.sigmoid(gi[:, _D:2 * _D] + gh[:, _D:2 * _D])
    n = jnp.tanh(gi[:, 2 * _D:] + r * gh[:, 2 * _D:])
    hn = (1.0 - z) * n + z * h
    hn_ref[...] = hn
    d = hn - h
    vn_ref[...] = _MOM * v_ref[...] + (1.0 - _MOM) * d * d


def _tc_gru(x, h_old, var_old, wih_t, whh_t, b_ih, b_hh):
    grid = (_B // _BM,)
    row_spec = pl.BlockSpec((_BM, _D), lambda i: (i, 0))
    full_w = pl.BlockSpec((_D, 3 * _D), lambda i: (0, 0))
    full_b = pl.BlockSpec((1, 3 * _D), lambda i: (0, 0))
    return pl.pallas_call(
        _gru_body,
        grid=grid,
        in_specs=[row_spec, row_spec, row_spec, full_w, full_w, full_b, full_b],
        out_specs=[row_spec, row_spec],
        out_shape=[
            jax.ShapeDtypeStruct((_B, _D), jnp.float32),
            jax.ShapeDtypeStruct((_B, _D), jnp.float32),
        ],
    )(x, h_old, var_old, wih_t, whh_t, b_ih, b_hh)


_CBLK = 5000


def _copy_body(h_ref, v_ref, out_ref):
    out_ref[0] = h_ref[...]
    out_ref[1] = v_ref[...]


def _tc_copy(hidden, variance):
    nb = _N // _CBLK
    blk = pl.BlockSpec((_CBLK, _D), lambda i: (i, 0))
    return pl.pallas_call(
        _copy_body,
        grid=(nb,),
        in_specs=[blk, blk],
        out_specs=pl.BlockSpec((2, _CBLK, _D), lambda i: (0, i, 0)),
        out_shape=jax.ShapeDtypeStruct((2, _N, _D), jnp.float32),
    )(hidden, variance)


def _sc_scatter_body(idx_hbm, hnew_hbm, varnew_hbm, stacked_hbm,
                     out_hbm, idx_v, win_v, dest_c, destn_c, src_c,
                     rowh_v, rowv_v, sem1, sem2):
    del stacked_hbm
    wid = lax.axis_index("c") * _NS + lax.axis_index("s")
    lo = wid * _ROWS_PW
    hi = jnp.minimum(lo + _ROWS_PW, _N)

    pltpu.sync_copy(idx_hbm, idx_v)
    iot = lax.iota(jnp.int32, 16)

    def init_body(i, _):
        win_v[pl.ds(i * 16, 16)] = jnp.full((16,), -1, jnp.int32)
        return 0

    lax.fori_loop(0, _MAP_VREGS, init_body, 0)

    def scan_body(c, _):
        chunk = idx_v[pl.ds(c * 16, 16)]
        k2 = chunk * 16 + iot
        pos = c * 16 + iot
        k2s, poss = plsc.sort_key_val(k2, pos)
        idxs = lax.shift_right_arithmetic(k2s, 4)
        nxt = _lane_gather(idxs, jnp.minimum(iot + 1, 15))
        bound = (idxs != nxt) | (iot == 15)
        inr = (idxs >= lo) & (idxs < hi)
        plsc.store_scatter(win_v, [idxs - lo], poss, mask=bound & inr)
        return 0

    lax.fori_loop(0, _B // 16, scan_body, 0)

    def compact_body(i, off):
        v = win_v[pl.ds(i * 16, 16)]
        m = v >= 0
        dvals = lo + i * 16 + iot
        plsc.store_compressed(dest_c.at[pl.ds(off, 16)], dvals, mask=m)
        plsc.store_compressed(destn_c.at[pl.ds(off, 16)], dvals + _N, mask=m)
        plsc.store_compressed(src_c.at[pl.ds(off, 16)], v, mask=m)
        return off + jnp.sum(m.astype(jnp.int32))

    k_cnt = lax.fori_loop(0, _MAP_VREGS, compact_body, 0)

    @pl.when(k_cnt > 0)
    def _pad():
        zz = jnp.zeros((16,), jnp.int32)
        dest_c[pl.ds(k_cnt, 16)] = _lane_gather(dest_c[pl.ds(0, 16)], zz)
        destn_c[pl.ds(k_cnt, 16)] = _lane_gather(destn_c[pl.ds(0, 16)], zz)
        src_c[pl.ds(k_cnt, 16)] = _lane_gather(src_c[pl.ds(0, 16)], zz)

    n_chunks = (k_cnt + 15) // 16

    def chunk_body(j, _):
        d16 = dest_c[pl.ds(j * 16, 16)]
        dn16 = destn_c[pl.ds(j * 16, 16)]
        s16 = src_c[pl.ds(j * 16, 16)]
        g1 = pltpu.async_copy(hnew_hbm.at[s16], rowh_v, sem1)
        g2 = pltpu.async_copy(varnew_hbm.at[s16], rowv_v, sem2)
        g1.wait()
        s1 = pltpu.async_copy(rowh_v, out_hbm.at[d16], sem1)
        g2.wait()
        s2 = pltpu.async_copy(rowv_v, out_hbm.at[dn16], sem2)
        s1.wait()
        s2.wait()
        return 0

    lax.fori_loop(0, n_chunks, chunk_body, 0)


_sc_scatter = _mpmd._mpmd_map(
    [(_mesh, _sc_scatter_body)],
    jax.ShapeDtypeStruct((2 * _N, _D), jnp.float32),
    input_output_aliases={3: 0},
    scratch_types=[
        pltpu.VMEM((_B,), jnp.int32),
        pltpu.VMEM((_MAP_PAD,), jnp.int32),
        pltpu.VMEM((_CAP,), jnp.int32),
        pltpu.VMEM((_CAP,), jnp.int32),
        pltpu.VMEM((_CAP,), jnp.int32),
        pltpu.VMEM((16, _D), jnp.float32),
        pltpu.VMEM((16, _D), jnp.float32),
        pltpu.SemaphoreType.DMA,
        pltpu.SemaphoreType.DMA,
    ],
    compiler_params=pltpu.CompilerParams(needs_layout_passes=False),
)


def kernel(x, idx, hidden, variance, W_ih, W_hh, b_ih, b_hh):
    idx = idx.astype(jnp.int32)
    h_old, var_old = _sc_gather(idx, hidden, variance)
    h_new, var_new = _tc_gru(
        x, h_old, var_old, W_ih.T, W_hh.T,
        b_ih.reshape(1, 3 * _D), b_hh.reshape(1, 3 * _D))
    stacked = _tc_copy(hidden, variance).reshape(2 * _N, _D)
    out2 = _sc_scatter(idx, h_new, var_new, stacked)
    return out2.reshape(2, _N, _D)


# speedup vs baseline: 20.1128x; 1.1384x over previous
import functools

import jax
import jax.numpy as jnp
from jax import lax
from jax.experimental import pallas as pl
from jax.experimental.pallas import tpu as pltpu
from jax.experimental.pallas import tpu_sc as plsc
from jax._src.pallas import mpmd as _mpmd

_N = 100000
_D = 128
_B = 16384
_MOM = 0.9

_NC = 2
_NS = 16
_NW = _NC * _NS
_BPW = _B // _NW
_ROWS_PW = 3128
_MAP_VREGS = (_ROWS_PW + 15) // 16
_MAP_PAD = _MAP_VREGS * 16
_CAP = ((_ROWS_PW + 127) // 128 + 1) * 128
_CHUNK = 128

_mesh = plsc.VectorSubcoreMesh(
    core_axis_name="c", subcore_axis_name="s", num_cores=_NC, num_subcores=_NS
)


def _wid():
    return lax.axis_index("c") * _NS + lax.axis_index("s")


def _lane_gather(x, i):
    dnums = lax.GatherDimensionNumbers(
        offset_dims=(), collapsed_slice_dims=(0,), start_index_map=(0,))
    return lax.gather(x, i[:, None], dnums, (1,),
                      mode=lax.GatherScatterMode.PROMISE_IN_BOUNDS)


@functools.partial(
    pl.kernel,
    out_type=(
        jax.ShapeDtypeStruct((_B, _D), jnp.float32),
        jax.ShapeDtypeStruct((_B, _D), jnp.float32),
        jax.ShapeDtypeStruct((_NW, _CAP), jnp.int32),
        jax.ShapeDtypeStruct((_NW, _CAP), jnp.int32),
        jax.ShapeDtypeStruct((_NW, 16), jnp.int32),
    ),
    mesh=_mesh,
    scratch_types=[
        pltpu.VMEM((_B,), jnp.int32),
        pltpu.VMEM((_MAP_PAD,), jnp.int32),
        pltpu.VMEM((_CAP,), jnp.int32),
        pltpu.VMEM((_CAP,), jnp.int32),
        pltpu.VMEM((16,), jnp.int32),
        pltpu.VMEM((_CHUNK, _D), jnp.float32),
        pltpu.VMEM((_CHUNK, _D), jnp.float32),
        pltpu.VMEM((_CHUNK, _D), jnp.float32),
        pltpu.VMEM((_CHUNK, _D), jnp.float32),
        pltpu.SemaphoreType.DMA,
        pltpu.SemaphoreType.DMA,
        pltpu.SemaphoreType.DMA,
        pltpu.SemaphoreType.DMA,
    ],
    compiler_params=pltpu.CompilerParams(needs_layout_passes=False),
)
def _sc_gather(idx_hbm, hidden_hbm, variance_hbm,
               hold_hbm, varold_hbm, dest_hbm, src_hbm, cnt_hbm,
               idx_v, win_v, dest_c, src_c, cnt_v,
               hb0, hb1, vb0, vb1, semh0, semh1, semv0, semv1):
    wid = _wid()
    base = wid * _BPW
    lo = wid * _ROWS_PW
    hi = jnp.minimum(lo + _ROWS_PW, _N)

    pltpu.sync_copy(idx_hbm, idx_v)

    def _sl(ch):
        return idx_v.at[pl.ds(base + ch * _CHUNK, _CHUNK)]

    gh0 = pltpu.async_copy(hidden_hbm.at[_sl(0)], hb0, semh0)
    gh1 = pltpu.async_copy(hidden_hbm.at[_sl(1)], hb1, semh1)
    gv0 = pltpu.async_copy(variance_hbm.at[_sl(0)], vb0, semv0)
    gv1 = pltpu.async_copy(variance_hbm.at[_sl(1)], vb1, semv1)

    iot = lax.iota(jnp.int32, 16)

    def init_body(i, _):
        win_v[pl.ds(i * 16, 16)] = jnp.full((16,), -1, jnp.int32)
        return 0

    lax.fori_loop(0, _MAP_VREGS, init_body, 0)

    def scan_body(c, _):
        chunk = idx_v[pl.ds(c * 16, 16)]
        k2 = chunk * 16 + iot
        pos = c * 16 + iot
        k2s, poss = plsc.sort_key_val(k2, pos)
        idxs = lax.shift_right_arithmetic(k2s, 4)
        nxt = _lane_gather(idxs, jnp.minimum(iot + 1, 15))
        bound = (idxs != nxt) | (iot == 15)
        inr = (idxs >= lo) & (idxs < hi)
        plsc.store_scatter(win_v, [idxs - lo], poss, mask=bound & inr)
        return 0

    lax.fori_loop(0, _B // 16, scan_body, 0)

    def compact_body(i, off):
        v = win_v[pl.ds(i * 16, 16)]
        m = v >= 0
        dvals = lo + i * 16 + iot
        plsc.store_compressed(dest_c.at[pl.ds(off, 16)], dvals, mask=m)
        plsc.store_compressed(src_c.at[pl.ds(off, 16)], v, mask=m)
        return off + jnp.sum(m.astype(jnp.int32))

    k_cnt = lax.fori_loop(0, _MAP_VREGS, compact_body, 0)

    @pl.when(k_cnt > 0)
    def _pad():
        zz = jnp.zeros((16,), jnp.int32)
        dpad = _lane_gather(dest_c[pl.ds(0, 16)], zz)
        spad = _lane_gather(src_c[pl.ds(0, 16)], zz)
        for t in range(_CHUNK // 16):
            dest_c[pl.ds(k_cnt + t * 16, 16)] = dpad
            src_c[pl.ds(k_cnt + t * 16, 16)] = spad

    cnt_v[...] = jnp.full((16,), k_cnt, jnp.int32)
    pltpu.sync_copy(dest_c, dest_hbm.at[wid])
    pltpu.sync_copy(src_c, src_hbm.at[wid])
    pltpu.sync_copy(cnt_v, cnt_hbm.at[wid])

    nch = _BPW // _CHUNK
    bufs = {"h": (hb0, hb1), "v": (vb0, vb1)}
    sems = {"h": (semh0, semh1), "v": (semv0, semv1)}
    outs = {"h": hold_hbm, "v": varold_hbm}
    srcs = {"h": hidden_hbm, "v": variance_hbm}
    pend = {"h": [gh0, gh1], "v": [gv0, gv1]}
    for ch in range(nch):
        for t in ("h", "v"):
            pend[t][ch % 2].wait()
            pltpu.sync_copy(bufs[t][ch % 2],
                            outs[t].at[pl.ds(base + ch * _CHUNK, _CHUNK)])
            if ch + 2 < nch:
                pend[t][ch % 2] = pltpu.async_copy(
                    srcs[t].at[_sl(ch + 2)], bufs[t][ch % 2], sems[t][ch % 2])


_BM = 1024


def _gru_body(x_ref, h_ref, v_ref, wih_ref, whh_ref, bih_ref, bhh_ref,
              hn_ref, vn_ref):
    x = x_ref[...]
    h = h_ref[...]
    gi = jnp.dot(x, wih_ref[...], preferred_element_type=jnp.float32) + bih_ref[...]
    gh = jnp.dot(h, whh_ref[...], preferred_element_type=jnp.float32) + bhh_ref[...]
    r = jax.nn.sigmoid(gi[:, :_D] + gh[:, :_D])
    z = jax.nn.sigmoid(gi[:, _D:2 * _D] + gh[:, _D:2 * _D])
    n = jnp.tanh(gi[:, 2 * _D:] + r * gh[:, 2 * _D:])
    hn = (1.0 - z) * n + z * h
    hn_ref[...] = hn
    d = hn - h
    vn_ref[...] = _MOM * v_ref[...] + (1.0 - _MOM) * d * d


def _tc_gru(x, h_old, var_old, wih_t, whh_t, b_ih, b_hh):
    grid = (_B // _BM,)
    row_spec = pl.BlockSpec((_BM, _D), lambda i: (i, 0))
    full_w = pl.BlockSpec((_D, 3 * _D), lambda i: (0, 0))
    full_b = pl.BlockSpec((1, 3 * _D), lambda i: (0, 0))
    return pl.pallas_call(
        _gru_body,
        grid=grid,
        in_specs=[row_spec, row_spec, row_spec, full_w, full_w, full_b, full_b],
        out_specs=[row_spec, row_spec],
        out_shape=[
            jax.ShapeDtypeStruct((_B, _D), jnp.float32),
            jax.ShapeDtypeStruct((_B, _D), jnp.float32),
        ],
    )(x, h_old, var_old, wih_t, whh_t, b_ih, b_hh)


_CBLK = 5000


def _copy_body(h_ref, v_ref, out_ref):
    out_ref[0] = h_ref[...]
    out_ref[1] = v_ref[...]


def _tc_copy(hidden, variance):
    nb = _N // _CBLK
    blk = pl.BlockSpec((_CBLK, _D), lambda i: (i, 0))
    return pl.pallas_call(
        _copy_body,
        grid=(nb,),
        in_specs=[blk, blk],
        out_specs=pl.BlockSpec((2, _CBLK, _D), lambda i: (0, i, 0)),
        out_shape=jax.ShapeDtypeStruct((2, _N, _D), jnp.float32),
    )(hidden, variance)


def _sc_scatter_body(hnew_hbm, varnew_hbm, dest_hbm, src_hbm, cnt_hbm,
                     stacked_hbm, out_hbm,
                     dest_c, src_c, cnt_v, rowh_v, rowv_v, sem1, sem2):
    del stacked_hbm
    wid = _wid()

    pltpu.sync_copy(dest_hbm.at[wid], dest_c)
    pltpu.sync_copy(src_hbm.at[wid], src_c)
    pltpu.sync_copy(cnt_hbm.at[wid], cnt_v)
    k_cnt = jnp.max(cnt_v[...], axis=0)
    n_chunks = (k_cnt + 15) // 16

    def chunk_body(j, _):
        d16 = dest_c[pl.ds(j * 16, 16)]
        dn16 = d16 + _N
        s16 = src_c[pl.ds(j * 16, 16)]
        g1 = pltpu.async_copy(hnew_hbm.at[s16], rowh_v, sem1)
        g2 = pltpu.async_copy(varnew_hbm.at[s16], rowv_v, sem2)
        g1.wait()
        s1 = pltpu.async_copy(rowh_v, out_hbm.at[d16], sem1)
        g2.wait()
        s2 = pltpu.async_copy(rowv_v, out_hbm.at[dn16], sem2)
        s1.wait()
        s2.wait()
        return 0

    lax.fori_loop(0, n_chunks, chunk_body, 0)


_sc_scatter = _mpmd._mpmd_map(
    [(_mesh, _sc_scatter_body)],
    jax.ShapeDtypeStruct((2 * _N, _D), jnp.float32),
    input_output_aliases={5: 0},
    scratch_types=[
        pltpu.VMEM((_CAP,), jnp.int32),
        pltpu.VMEM((_CAP,), jnp.int32),
        pltpu.VMEM((16,), jnp.int32),
        pltpu.VMEM((16, _D), jnp.float32),
        pltpu.VMEM((16, _D), jnp.float32),
        pltpu.SemaphoreType.DMA,
        pltpu.SemaphoreType.DMA,
    ],
    compiler_params=pltpu.CompilerParams(needs_layout_passes=False),
)


def kernel(x, idx, hidden, variance, W_ih, W_hh, b_ih, b_hh):
    idx = idx.astype(jnp.int32)
    h_old, var_old, dest, src, cnt = _sc_gather(idx, hidden, variance)
    h_new, var_new = _tc_gru(
        x, h_old, var_old, W_ih.T, W_hh.T,
        b_ih.reshape(1, 3 * _D), b_hh.reshape(1, 3 * _D))
    stacked = _tc_copy(hidden, variance).reshape(2 * _N, _D)
    out2 = _sc_scatter(h_new, var_new, dest, src, cnt, stacked)
    return out2.reshape(2, _N, _D)


# speedup vs baseline: 20.1305x; 1.0009x over previous
import functools

import jax
import jax.numpy as jnp
from jax import lax
from jax.experimental import pallas as pl
from jax.experimental.pallas import tpu as pltpu
from jax.experimental.pallas import tpu_sc as plsc
from jax._src.pallas import mpmd as _mpmd

_N = 100000
_D = 128
_B = 16384
_MOM = 0.9

_NC = 2
_NS = 16
_NW = _NC * _NS
_BPW = _B // _NW
_ROWS_PW = 3128
_MAP_VREGS = (_ROWS_PW + 15) // 16
_MAP_PAD = _MAP_VREGS * 16
_CAP = ((_ROWS_PW + 127) // 128 + 1) * 128
_CHUNK = 128

_mesh = plsc.VectorSubcoreMesh(
    core_axis_name="c", subcore_axis_name="s", num_cores=_NC, num_subcores=_NS
)


def _wid():
    return lax.axis_index("c") * _NS + lax.axis_index("s")


def _lane_gather(x, i):
    dnums = lax.GatherDimensionNumbers(
        offset_dims=(), collapsed_slice_dims=(0,), start_index_map=(0,))
    return lax.gather(x, i[:, None], dnums, (1,),
                      mode=lax.GatherScatterMode.PROMISE_IN_BOUNDS)


@functools.partial(
    pl.kernel,
    out_type=(
        jax.ShapeDtypeStruct((_B, _D), jnp.float32),
        jax.ShapeDtypeStruct((_B, _D), jnp.float32),
        jax.ShapeDtypeStruct((_NW, _CAP), jnp.int32),
        jax.ShapeDtypeStruct((_NW, _CAP), jnp.int32),
        jax.ShapeDtypeStruct((_NW, 16), jnp.int32),
    ),
    mesh=_mesh,
    scratch_types=[
        pltpu.VMEM((_B,), jnp.int32),
        pltpu.VMEM((_MAP_PAD,), jnp.int32),
        pltpu.VMEM((_CAP,), jnp.int32),
        pltpu.VMEM((_CAP,), jnp.int32),
        pltpu.VMEM((16,), jnp.int32),
        pltpu.VMEM((_CHUNK, _D), jnp.float32),
        pltpu.VMEM((_CHUNK, _D), jnp.float32),
        pltpu.VMEM((_CHUNK, _D), jnp.float32),
        pltpu.VMEM((_CHUNK, _D), jnp.float32),
        pltpu.SemaphoreType.DMA,
        pltpu.SemaphoreType.DMA,
        pltpu.SemaphoreType.DMA,
        pltpu.SemaphoreType.DMA,
    ],
    compiler_params=pltpu.CompilerParams(needs_layout_passes=False),
)
def _sc_gather(idx_hbm, hidden_hbm, variance_hbm,
               hold_hbm, varold_hbm, dest_hbm, src_hbm, cnt_hbm,
               idx_v, win_v, dest_c, src_c, cnt_v,
               hb0, hb1, vb0, vb1, semh0, semh1, semv0, semv1):
    wid = _wid()
    base = wid * _BPW
    lo = wid * _ROWS_PW
    hi = jnp.minimum(lo + _ROWS_PW, _N)

    pltpu.sync_copy(idx_hbm, idx_v)

    def _sl(ch):
        return idx_v.at[pl.ds(base + ch * _CHUNK, _CHUNK)]

    gh0 = pltpu.async_copy(hidden_hbm.at[_sl(0)], hb0, semh0)
    gh1 = pltpu.async_copy(hidden_hbm.at[_sl(1)], hb1, semh1)
    gv0 = pltpu.async_copy(variance_hbm.at[_sl(0)], vb0, semv0)
    gv1 = pltpu.async_copy(variance_hbm.at[_sl(1)], vb1, semv1)

    iot = lax.iota(jnp.int32, 16)

    def init_body(i, _):
        win_v[pl.ds(i * 16, 16)] = jnp.full((16,), -1, jnp.int32)
        return 0

    lax.fori_loop(0, _MAP_VREGS, init_body, 0)

    def scan_body(c, _):
        chunk = idx_v[pl.ds(c * 16, 16)]
        k2 = chunk * 16 + iot
        pos = c * 16 + iot
        k2s, poss = plsc.sort_key_val(k2, pos)
        idxs = lax.shift_right_arithmetic(k2s, 4)
        nxt = _lane_gather(idxs, jnp.minimum(iot + 1, 15))
        bound = (idxs != nxt) | (iot == 15)
        inr = (idxs >= lo) & (idxs < hi)
        plsc.store_scatter(win_v, [idxs - lo], poss, mask=bound & inr)
        return 0

    lax.fori_loop(0, _B // 16, scan_body, 0)

    def compact_body(i, off):
        v = win_v[pl.ds(i * 16, 16)]
        m = v >= 0
        dvals = lo + i * 16 + iot
        plsc.store_compressed(dest_c.at[pl.ds(off, 16)], dvals, mask=m)
        plsc.store_compressed(src_c.at[pl.ds(off, 16)], v, mask=m)
        return off + jnp.sum(m.astype(jnp.int32))

    k_cnt = lax.fori_loop(0, _MAP_VREGS, compact_body, 0)

    @pl.when(k_cnt > 0)
    def _pad():
        zz = jnp.zeros((16,), jnp.int32)
        dpad = _lane_gather(dest_c[pl.ds(0, 16)], zz)
        spad = _lane_gather(src_c[pl.ds(0, 16)], zz)
        for t in range(_CHUNK // 16):
            dest_c[pl.ds(k_cnt + t * 16, 16)] = dpad
            src_c[pl.ds(k_cnt + t * 16, 16)] = spad

    cnt_v[...] = jnp.full((16,), k_cnt, jnp.int32)
    pltpu.sync_copy(dest_c, dest_hbm.at[wid])
    pltpu.sync_copy(src_c, src_hbm.at[wid])
    pltpu.sync_copy(cnt_v, cnt_hbm.at[wid])

    nch = _BPW // _CHUNK
    bufs = {"h": (hb0, hb1), "v": (vb0, vb1)}
    sems = {"h": (semh0, semh1), "v": (semv0, semv1)}
    outs = {"h": hold_hbm, "v": varold_hbm}
    srcs = {"h": hidden_hbm, "v": variance_hbm}
    pend = {"h": [gh0, gh1], "v": [gv0, gv1]}
    for ch in range(nch):
        for t in ("h", "v"):
            pend[t][ch % 2].wait()
            pltpu.sync_copy(bufs[t][ch % 2],
                            outs[t].at[pl.ds(base + ch * _CHUNK, _CHUNK)])
            if ch + 2 < nch:
                pend[t][ch % 2] = pltpu.async_copy(
                    srcs[t].at[_sl(ch + 2)], bufs[t][ch % 2], sems[t][ch % 2])


_BM = 1024


def _gru_body(x_ref, h_ref, v_ref, wih_ref, whh_ref, bih_ref, bhh_ref,
              hn_ref, vn_ref):
    x = x_ref[...]
    h = h_ref[...]
    gi = jnp.dot(x, wih_ref[...], preferred_element_type=jnp.float32) + bih_ref[...]
    gh = jnp.dot(h, whh_ref[...], preferred_element_type=jnp.float32) + bhh_ref[...]
    r = jax.nn.sigmoid(gi[:, :_D] + gh[:, :_D])
    z = jax.nn.sigmoid(gi[:, _D:2 * _D] + gh[:, _D:2 * _D])
    n = jnp.tanh(gi[:, 2 * _D:] + r * gh[:, 2 * _D:])
    hn = (1.0 - z) * n + z * h
    hn_ref[...] = hn
    d = hn - h
    vn_ref[...] = _MOM * v_ref[...] + (1.0 - _MOM) * d * d


def _tc_gru(x, h_old, var_old, wih_t, whh_t, b_ih, b_hh):
    grid = (_B // _BM,)
    row_spec = pl.BlockSpec((_BM, _D), lambda i: (i, 0))
    full_w = pl.BlockSpec((_D, 3 * _D), lambda i: (0, 0))
    full_b = pl.BlockSpec((1, 3 * _D), lambda i: (0, 0))
    return pl.pallas_call(
        _gru_body,
        grid=grid,
        in_specs=[row_spec, row_spec, row_spec, full_w, full_w, full_b, full_b],
        out_specs=[row_spec, row_spec],
        out_shape=[
            jax.ShapeDtypeStruct((_B, _D), jnp.float32),
            jax.ShapeDtypeStruct((_B, _D), jnp.float32),
        ],
    )(x, h_old, var_old, wih_t, whh_t, b_ih, b_hh)


_CBLK = 5000


def _copy_body(h_ref, v_ref, out_ref):
    out_ref[0] = h_ref[...]
    out_ref[1] = v_ref[...]


def _tc_copy(hidden, variance):
    nb = _N // _CBLK
    blk = pl.BlockSpec((_CBLK, _D), lambda i: (i, 0))
    return pl.pallas_call(
        _copy_body,
        grid=(nb,),
        in_specs=[blk, blk],
        out_specs=pl.BlockSpec((2, _CBLK, _D), lambda i: (0, i, 0)),
        out_shape=jax.ShapeDtypeStruct((2, _N, _D), jnp.float32),
    )(hidden, variance)


def _sc_scatter_body(hnew_hbm, varnew_hbm, dest_hbm, src_hbm, cnt_hbm,
                     stacked_hbm, out_hbm,
                     dest_c, src_c, cnt_v, rowh_v, rowv_v, sem1, sem2):
    del stacked_hbm
    wid = _wid()

    pltpu.sync_copy(dest_hbm.at[wid], dest_c)
    pltpu.sync_copy(src_hbm.at[wid], src_c)
    pltpu.sync_copy(cnt_hbm.at[wid], cnt_v)
    k_cnt = jnp.max(cnt_v[...], axis=0)
    n_chunks = (k_cnt + 15) // 16

    def chunk_body(j, _):
        d16 = dest_c[pl.ds(j * 16, 16)]
        dn16 = d16 + _N
        s16 = src_c[pl.ds(j * 16, 16)]
        g1 = pltpu.async_copy(hnew_hbm.at[s16], rowh_v, sem1)
        g2 = pltpu.async_copy(varnew_hbm.at[s16], rowv_v, sem2)
        g1.wait()
        s1 = pltpu.async_copy(rowh_v, out_hbm.at[d16], sem1)
        g2.wait()
        s2 = pltpu.async_copy(rowv_v, out_hbm.at[dn16], sem2)
        s1.wait()
        s2.wait()
        return 0

    lax.fori_loop(0, n_chunks, chunk_body, 0)


_sc_scatter = _mpmd._mpmd_map(
    [(_mesh, _sc_scatter_body)],
    jax.ShapeDtypeStruct((2 * _N, _D), jnp.float32),
    input_output_aliases={5: 0},
    scratch_types=[
        pltpu.VMEM((_CAP,), jnp.int32),
        pltpu.VMEM((_CAP,), jnp.int32),
        pltpu.VMEM((16,), jnp.int32),
        pltpu.VMEM((16, _D), jnp.float32),
        pltpu.VMEM((16, _D), jnp.float32),
        pltpu.SemaphoreType.DMA,
        pltpu.SemaphoreType.DMA,
    ],
    compiler_params=pltpu.CompilerParams(needs_layout_passes=False),
)


def kernel(x, idx, hidden, variance, W_ih, W_hh, b_ih, b_hh):
    idx = idx.astype(jnp.int32)
    h_old, var_old, dest, src, cnt = _sc_gather(idx, hidden, variance)
    stacked = _tc_copy(hidden, variance).reshape(2 * _N, _D)
    h_new, var_new = _tc_gru(
        x, h_old, var_old, W_ih.T, W_hh.T,
        b_ih.reshape(1, 3 * _D), b_hh.reshape(1, 3 * _D))
    out2 = _sc_scatter(h_new, var_new, dest, src, cnt, stacked)
    return out2.reshape(2, _N, _D)


# speedup vs baseline: 20.9843x; 1.0424x over previous
import functools

import jax
import jax.numpy as jnp
from jax import lax
from jax.experimental import pallas as pl
from jax.experimental.pallas import tpu as pltpu
from jax.experimental.pallas import tpu_sc as plsc
from jax._src.pallas import mpmd as _mpmd

_N = 100000
_D = 128
_B = 16384
_MOM = 0.9

_NC = 2
_NS = 16
_NW = _NC * _NS
_BPW = _B // _NW
_ROWS_PW = 3128
_MAP_VREGS = (_ROWS_PW + 15) // 16
_MAP_PAD = _MAP_VREGS * 16
_CAP = ((_ROWS_PW + 127) // 128 + 1) * 128
_CHUNK = 128

_mesh = plsc.VectorSubcoreMesh(
    core_axis_name="c", subcore_axis_name="s", num_cores=_NC, num_subcores=_NS
)


def _wid():
    return lax.axis_index("c") * _NS + lax.axis_index("s")


def _lane_gather(x, i):
    dnums = lax.GatherDimensionNumbers(
        offset_dims=(), collapsed_slice_dims=(0,), start_index_map=(0,))
    return lax.gather(x, i[:, None], dnums, (1,),
                      mode=lax.GatherScatterMode.PROMISE_IN_BOUNDS)


@functools.partial(
    pl.kernel,
    out_type=(
        jax.ShapeDtypeStruct((_B, _D), jnp.float32),
        jax.ShapeDtypeStruct((_B, _D), jnp.float32),
        jax.ShapeDtypeStruct((_NW, _CAP), jnp.int32),
        jax.ShapeDtypeStruct((_NW, _CAP), jnp.int32),
        jax.ShapeDtypeStruct((_NW, 16), jnp.int32),
    ),
    mesh=_mesh,
    scratch_types=[
        pltpu.VMEM((_B,), jnp.int32),
        pltpu.VMEM((_MAP_PAD,), jnp.int32),
        pltpu.VMEM((_CAP,), jnp.int32),
        pltpu.VMEM((_CAP,), jnp.int32),
        pltpu.VMEM((16,), jnp.int32),
        pltpu.VMEM((_CHUNK, _D), jnp.float32),
        pltpu.VMEM((_CHUNK, _D), jnp.float32),
        pltpu.VMEM((_CHUNK, _D), jnp.float32),
        pltpu.VMEM((_CHUNK, _D), jnp.float32),
        pltpu.SemaphoreType.DMA,
        pltpu.SemaphoreType.DMA,
        pltpu.SemaphoreType.DMA,
        pltpu.SemaphoreType.DMA,
    ],
    compiler_params=pltpu.CompilerParams(needs_layout_passes=False),
)
def _sc_gather(idx_hbm, hidden_hbm, variance_hbm,
               hold_hbm, varold_hbm, dest_hbm, src_hbm, cnt_hbm,
               idx_v, win_v, dest_c, src_c, cnt_v,
               hb0, hb1, vb0, vb1, semh0, semh1, semv0, semv1):
    wid = _wid()
    base = wid * _BPW
    lo = wid * _ROWS_PW
    hi = jnp.minimum(lo + _ROWS_PW, _N)

    pltpu.sync_copy(idx_hbm, idx_v)

    def _sl(ch):
        return idx_v.at[pl.ds(base + ch * _CHUNK, _CHUNK)]

    gh0 = pltpu.async_copy(hidden_hbm.at[_sl(0)], hb0, semh0)
    gh1 = pltpu.async_copy(hidden_hbm.at[_sl(1)], hb1, semh1)
    gv0 = pltpu.async_copy(variance_hbm.at[_sl(0)], vb0, semv0)
    gv1 = pltpu.async_copy(variance_hbm.at[_sl(1)], vb1, semv1)

    iot = lax.iota(jnp.int32, 16)

    def init_body(i, _):
        win_v[pl.ds(i * 16, 16)] = jnp.full((16,), -1, jnp.int32)
        return 0

    lax.fori_loop(0, _MAP_VREGS, init_body, 0)

    def scan_body(c, _):
        for u in range(2):
            cc = c * 2 + u
            chunk = idx_v[pl.ds(cc * 16, 16)]
            k2 = chunk * 16 + iot
            pos = cc * 16 + iot
            k2s, poss = plsc.sort_key_val(k2, pos)
            idxs = lax.shift_right_arithmetic(k2s, 4)
            nxt = _lane_gather(idxs, jnp.minimum(iot + 1, 15))
            bound = (idxs != nxt) | (iot == 15)
            inr = (idxs >= lo) & (idxs < hi)
            plsc.store_scatter(win_v, [idxs - lo], poss, mask=bound & inr)
        return 0

    lax.fori_loop(0, _B // 32, scan_body, 0)

    def compact_body(i, off):
        v = win_v[pl.ds(i * 16, 16)]
        m = v >= 0
        dvals = lo + i * 16 + iot
        plsc.store_compressed(dest_c.at[pl.ds(off, 16)], dvals, mask=m)
        plsc.store_compressed(src_c.at[pl.ds(off, 16)], v, mask=m)
        return off + jnp.sum(m.astype(jnp.int32))

    k_cnt = lax.fori_loop(0, _MAP_VREGS, compact_body, 0)

    @pl.when(k_cnt > 0)
    def _pad():
        zz = jnp.zeros((16,), jnp.int32)
        dpad = _lane_gather(dest_c[pl.ds(0, 16)], zz)
        spad = _lane_gather(src_c[pl.ds(0, 16)], zz)
        for t in range(_CHUNK // 16):
            dest_c[pl.ds(k_cnt + t * 16, 16)] = dpad
            src_c[pl.ds(k_cnt + t * 16, 16)] = spad

    cnt_v[...] = jnp.full((16,), k_cnt, jnp.int32)
    pltpu.sync_copy(dest_c, dest_hbm.at[wid])
    pltpu.sync_copy(src_c, src_hbm.at[wid])
    pltpu.sync_copy(cnt_v, cnt_hbm.at[wid])

    nch = _BPW // _CHUNK
    bufs = {"h": (hb0, hb1), "v": (vb0, vb1)}
    sems = {"h": (semh0, semh1), "v": (semv0, semv1)}
    outs = {"h": hold_hbm, "v": varold_hbm}
    srcs = {"h": hidden_hbm, "v": variance_hbm}
    pend = {"h": [gh0, gh1], "v": [gv0, gv1]}
    for ch in range(nch):
        for t in ("h", "v"):
            pend[t][ch % 2].wait()
            pltpu.sync_copy(bufs[t][ch % 2],
                            outs[t].at[pl.ds(base + ch * _CHUNK, _CHUNK)])
            if ch + 2 < nch:
                pend[t][ch % 2] = pltpu.async_copy(
                    srcs[t].at[_sl(ch + 2)], bufs[t][ch % 2], sems[t][ch % 2])


_BM = 1024


def _gru_body(x_ref, h_ref, v_ref, wih_ref, whh_ref, bih_ref, bhh_ref,
              hn_ref, vn_ref):
    x = x_ref[...]
    h = h_ref[...]
    gi = jnp.dot(x, wih_ref[...], preferred_element_type=jnp.float32) + bih_ref[...]
    gh = jnp.dot(h, whh_ref[...], preferred_element_type=jnp.float32) + bhh_ref[...]
    r = jax.nn.sigmoid(gi[:, :_D] + gh[:, :_D])
    z = jax.nn.sigmoid(gi[:, _D:2 * _D] + gh[:, _D:2 * _D])
    n = jnp.tanh(gi[:, 2 * _D:] + r * gh[:, 2 * _D:])
    hn = (1.0 - z) * n + z * h
    hn_ref[...] = hn
    d = hn - h
    vn_ref[...] = _MOM * v_ref[...] + (1.0 - _MOM) * d * d


def _tc_gru(x, h_old, var_old, wih_t, whh_t, b_ih, b_hh):
    grid = (_B // _BM,)
    row_spec = pl.BlockSpec((_BM, _D), lambda i: (i, 0))
    full_w = pl.BlockSpec((_D, 3 * _D), lambda i: (0, 0))
    full_b = pl.BlockSpec((1, 3 * _D), lambda i: (0, 0))
    return pl.pallas_call(
        _gru_body,
        grid=grid,
        in_specs=[row_spec, row_spec, row_spec, full_w, full_w, full_b, full_b],
        out_specs=[row_spec, row_spec],
        out_shape=[
            jax.ShapeDtypeStruct((_B, _D), jnp.float32),
            jax.ShapeDtypeStruct((_B, _D), jnp.float32),
        ],
    )(x, h_old, var_old, wih_t, whh_t, b_ih, b_hh)


_CBLK = 5000


def _copy_body(h_ref, v_ref, out_ref):
    out_ref[0] = h_ref[...]
    out_ref[1] = v_ref[...]


def _tc_copy(hidden, variance):
    nb = _N // _CBLK
    blk = pl.BlockSpec((_CBLK, _D), lambda i: (i, 0))
    return pl.pallas_call(
        _copy_body,
        grid=(nb,),
        in_specs=[blk, blk],
        out_specs=pl.BlockSpec((2, _CBLK, _D), lambda i: (0, i, 0)),
        out_shape=jax.ShapeDtypeStruct((2, _N, _D), jnp.float32),
    )(hidden, variance)


def _sc_scatter_body(hnew_hbm, varnew_hbm, dest_hbm, src_hbm, cnt_hbm,
                     stacked_hbm, out_hbm,
                     dest_c, src_c, cnt_v, rowh_v, rowv_v, sem1, sem2):
    del stacked_hbm
    wid = _wid()

    pltpu.sync_copy(dest_hbm.at[wid], dest_c)
    pltpu.sync_copy(src_hbm.at[wid], src_c)
    pltpu.sync_copy(cnt_hbm.at[wid], cnt_v)
    k_cnt = jnp.max(cnt_v[...], axis=0)
    n_chunks = (k_cnt + _CHUNK - 1) // _CHUNK

    def chunk_body(j, _):
        gs, ss = [], []
        for t in range(_CHUNK // 16):
            s16 = src_c[pl.ds(j * _CHUNK + t * 16, 16)]
            gs.append(pltpu.async_copy(
                hnew_hbm.at[s16], rowh_v.at[pl.ds(t * 16, 16)], sem1))
            gs.append(pltpu.async_copy(
                varnew_hbm.at[s16], rowv_v.at[pl.ds(t * 16, 16)], sem2))
        for g in gs:
            g.wait()
        for t in range(_CHUNK // 16):
            d16 = dest_c[pl.ds(j * _CHUNK + t * 16, 16)]
            ss.append(pltpu.async_copy(
                rowh_v.at[pl.ds(t * 16, 16)], out_hbm.at[d16], sem1))
            ss.append(pltpu.async_copy(
                rowv_v.at[pl.ds(t * 16, 16)], out_hbm.at[d16 + _N], sem2))
        for s in ss:
            s.wait()
        return 0

    lax.fori_loop(0, n_chunks, chunk_body, 0)


_sc_scatter = _mpmd._mpmd_map(
    [(_mesh, _sc_scatter_body)],
    jax.ShapeDtypeStruct((2 * _N, _D), jnp.float32),
    input_output_aliases={5: 0},
    scratch_types=[
        pltpu.VMEM((_CAP,), jnp.int32),
        pltpu.VMEM((_CAP,), jnp.int32),
        pltpu.VMEM((16,), jnp.int32),
        pltpu.VMEM((_CHUNK, _D), jnp.float32),
        pltpu.VMEM((_CHUNK, _D), jnp.float32),
        pltpu.SemaphoreType.DMA,
        pltpu.SemaphoreType.DMA,
    ],
    compiler_params=pltpu.CompilerParams(needs_layout_passes=False),
)


def kernel(x, idx, hidden, variance, W_ih, W_hh, b_ih, b_hh):
    idx = idx.astype(jnp.int32)
    h_old, var_old, dest, src, cnt = _sc_gather(idx, hidden, variance)
    stacked = _tc_copy(hidden, variance).reshape(2 * _N, _D)
    h_new, var_new = _tc_gru(
        x, h_old, var_old, W_ih.T, W_hh.T,
        b_ih.reshape(1, 3 * _D), b_hh.reshape(1, 3 * _D))
    out2 = _sc_scatter(h_new, var_new, dest, src, cnt, stacked)
    return out2.reshape(2, _N, _D)


# speedup vs baseline: 21.1284x; 1.0069x over previous
import functools

import jax
import jax.numpy as jnp
from jax import lax
from jax.experimental import pallas as pl
from jax.experimental.pallas import tpu as pltpu
from jax.experimental.pallas import tpu_sc as plsc
from jax._src.pallas import mpmd as _mpmd

_N = 100000
_D = 128
_B = 16384
_MOM = 0.9

_NC = 2
_NS = 16
_NW = _NC * _NS
_BPW = _B // _NW
_ROWS_PW = 3128
_MAP_VREGS = (_ROWS_PW + 15) // 16
_MAP_PAD = _MAP_VREGS * 16
_CAP = ((_ROWS_PW + 127) // 128 + 1) * 128
_CHUNK = 128

_mesh = plsc.VectorSubcoreMesh(
    core_axis_name="c", subcore_axis_name="s", num_cores=_NC, num_subcores=_NS
)


def _wid():
    return lax.axis_index("c") * _NS + lax.axis_index("s")


def _lane_gather(x, i):
    dnums = lax.GatherDimensionNumbers(
        offset_dims=(), collapsed_slice_dims=(0,), start_index_map=(0,))
    return lax.gather(x, i[:, None], dnums, (1,),
                      mode=lax.GatherScatterMode.PROMISE_IN_BOUNDS)


@functools.partial(
    pl.kernel,
    out_type=(
        jax.ShapeDtypeStruct((_B, _D), jnp.float32),
        jax.ShapeDtypeStruct((_B, _D), jnp.float32),
        jax.ShapeDtypeStruct((_NW, _CAP), jnp.int32),
        jax.ShapeDtypeStruct((_NW, _CAP), jnp.int32),
        jax.ShapeDtypeStruct((_NW, 16), jnp.int32),
    ),
    mesh=_mesh,
    scratch_types=[
        pltpu.VMEM((_B,), jnp.int32),
        pltpu.VMEM((_MAP_PAD,), jnp.int32),
        pltpu.VMEM((_CAP,), jnp.int32),
        pltpu.VMEM((_CAP,), jnp.int32),
        pltpu.VMEM((16,), jnp.int32),
        pltpu.VMEM((_CHUNK, _D), jnp.float32),
        pltpu.VMEM((_CHUNK, _D), jnp.float32),
        pltpu.VMEM((_CHUNK, _D), jnp.float32),
        pltpu.VMEM((_CHUNK, _D), jnp.float32),
        pltpu.SemaphoreType.DMA,
        pltpu.SemaphoreType.DMA,
        pltpu.SemaphoreType.DMA,
        pltpu.SemaphoreType.DMA,
    ],
    compiler_params=pltpu.CompilerParams(needs_layout_passes=False),
    cost_estimate=pl.CostEstimate(
        flops=2_000_000, bytes_accessed=40_000_000, transcendentals=0),
)
def _sc_gather(idx_hbm, hidden_hbm, variance_hbm,
               hold_hbm, varold_hbm, dest_hbm, src_hbm, cnt_hbm,
               idx_v, win_v, dest_c, src_c, cnt_v,
               hb0, hb1, vb0, vb1, semh0, semh1, semv0, semv1):
    wid = _wid()
    base = wid * _BPW
    lo = wid * _ROWS_PW
    hi = jnp.minimum(lo + _ROWS_PW, _N)

    pltpu.sync_copy(idx_hbm, idx_v)

    def _sl(ch):
        return idx_v.at[pl.ds(base + ch * _CHUNK, _CHUNK)]

    gh0 = pltpu.async_copy(hidden_hbm.at[_sl(0)], hb0, semh0)
    gh1 = pltpu.async_copy(hidden_hbm.at[_sl(1)], hb1, semh1)
    gv0 = pltpu.async_copy(variance_hbm.at[_sl(0)], vb0, semv0)
    gv1 = pltpu.async_copy(variance_hbm.at[_sl(1)], vb1, semv1)

    iot = lax.iota(jnp.int32, 16)

    def init_body(i, _):
        win_v[pl.ds(i * 16, 16)] = jnp.full((16,), -1, jnp.int32)
        return 0

    lax.fori_loop(0, _MAP_VREGS, init_body, 0)

    def scan_body(c, _):
        for u in range(2):
            cc = c * 2 + u
            chunk = idx_v[pl.ds(cc * 16, 16)]
            k2 = chunk * 16 + iot
            pos = cc * 16 + iot
            k2s, poss = plsc.sort_key_val(k2, pos)
            idxs = lax.shift_right_arithmetic(k2s, 4)
            nxt = _lane_gather(idxs, jnp.minimum(iot + 1, 15))
            bound = (idxs != nxt) | (iot == 15)
            inr = (idxs >= lo) & (idxs < hi)
            plsc.store_scatter(win_v, [idxs - lo], poss, mask=bound & inr)
        return 0

    lax.fori_loop(0, _B // 32, scan_body, 0)

    def compact_body(i, off):
        v = win_v[pl.ds(i * 16, 16)]
        m = v >= 0
        dvals = lo + i * 16 + iot
        plsc.store_compressed(dest_c.at[pl.ds(off, 16)], dvals, mask=m)
        plsc.store_compressed(src_c.at[pl.ds(off, 16)], v, mask=m)
        return off + jnp.sum(m.astype(jnp.int32))

    k_cnt = lax.fori_loop(0, _MAP_VREGS, compact_body, 0)

    @pl.when(k_cnt > 0)
    def _pad():
        zz = jnp.zeros((16,), jnp.int32)
        dpad = _lane_gather(dest_c[pl.ds(0, 16)], zz)
        spad = _lane_gather(src_c[pl.ds(0, 16)], zz)
        for t in range(_CHUNK // 16):
            dest_c[pl.ds(k_cnt + t * 16, 16)] = dpad
            src_c[pl.ds(k_cnt + t * 16, 16)] = spad

    cnt_v[...] = jnp.full((16,), k_cnt, jnp.int32)
    pltpu.sync_copy(dest_c, dest_hbm.at[wid])
    pltpu.sync_copy(src_c, src_hbm.at[wid])
    pltpu.sync_copy(cnt_v, cnt_hbm.at[wid])

    nch = _BPW // _CHUNK
    bufs = {"h": (hb0, hb1), "v": (vb0, vb1)}
    sems = {"h": (semh0, semh1), "v": (semv0, semv1)}
    outs = {"h": hold_hbm, "v": varold_hbm}
    srcs = {"h": hidden_hbm, "v": variance_hbm}
    pend = {"h": [gh0, gh1], "v": [gv0, gv1]}
    for ch in range(nch):
        for t in ("h", "v"):
            pend[t][ch % 2].wait()
            pltpu.sync_copy(bufs[t][ch % 2],
                            outs[t].at[pl.ds(base + ch * _CHUNK, _CHUNK)])
            if ch + 2 < nch:
                pend[t][ch % 2] = pltpu.async_copy(
                    srcs[t].at[_sl(ch + 2)], bufs[t][ch % 2], sems[t][ch % 2])


_BM = 1024


def _gru_body(x_ref, h_ref, v_ref, wih_ref, whh_ref, bih_ref, bhh_ref,
              hn_ref, vn_ref):
    x = x_ref[...]
    h = h_ref[...]
    gi = jnp.dot(x, wih_ref[...], preferred_element_type=jnp.float32) + bih_ref[...]
    gh = jnp.dot(h, whh_ref[...], preferred_element_type=jnp.float32) + bhh_ref[...]
    r = jax.nn.sigmoid(gi[:, :_D] + gh[:, :_D])
    z = jax.nn.sigmoid(gi[:, _D:2 * _D] + gh[:, _D:2 * _D])
    n = jnp.tanh(gi[:, 2 * _D:] + r * gh[:, 2 * _D:])
    hn = (1.0 - z) * n + z * h
    hn_ref[...] = hn
    d = hn - h
    vn_ref[...] = _MOM * v_ref[...] + (1.0 - _MOM) * d * d


def _tc_gru(x, h_old, var_old, wih_t, whh_t, b_ih, b_hh):
    grid = (_B // _BM,)
    row_spec = pl.BlockSpec((_BM, _D), lambda i: (i, 0))
    full_w = pl.BlockSpec((_D, 3 * _D), lambda i: (0, 0))
    full_b = pl.BlockSpec((1, 3 * _D), lambda i: (0, 0))
    return pl.pallas_call(
        _gru_body,
        grid=grid,
        in_specs=[row_spec, row_spec, row_spec, full_w, full_w, full_b, full_b],
        out_specs=[row_spec, row_spec],
        out_shape=[
            jax.ShapeDtypeStruct((_B, _D), jnp.float32),
            jax.ShapeDtypeStruct((_B, _D), jnp.float32),
        ],
    )(x, h_old, var_old, wih_t, whh_t, b_ih, b_hh)


_CBLK = 5000


def _copy_body(h_ref, v_ref, out_ref):
    out_ref[0] = h_ref[...]
    out_ref[1] = v_ref[...]


def _tc_copy(hidden, variance):
    nb = _N // _CBLK
    blk = pl.BlockSpec((_CBLK, _D), lambda i: (i, 0))
    return pl.pallas_call(
        _copy_body,
        grid=(nb,),
        in_specs=[blk, blk],
        out_specs=pl.BlockSpec((2, _CBLK, _D), lambda i: (0, i, 0)),
        out_shape=jax.ShapeDtypeStruct((2, _N, _D), jnp.float32),
    )(hidden, variance)


def _sc_scatter_body(hnew_hbm, varnew_hbm, dest_hbm, src_hbm, cnt_hbm,
                     stacked_hbm, out_hbm,
                     dest_c, src_c, cnt_v, rowh_v, rowv_v, sem1, sem2):
    del stacked_hbm
    wid = _wid()

    pltpu.sync_copy(dest_hbm.at[wid], dest_c)
    pltpu.sync_copy(src_hbm.at[wid], src_c)
    pltpu.sync_copy(cnt_hbm.at[wid], cnt_v)
    k_cnt = jnp.max(cnt_v[...], axis=0)
    n_chunks = (k_cnt + _CHUNK - 1) // _CHUNK

    def chunk_body(j, _):
        gs, ss = [], []
        for t in range(_CHUNK // 16):
            s16 = src_c[pl.ds(j * _CHUNK + t * 16, 16)]
            gs.append(pltpu.async_copy(
                hnew_hbm.at[s16], rowh_v.at[pl.ds(t * 16, 16)], sem1))
            gs.append(pltpu.async_copy(
                varnew_hbm.at[s16], rowv_v.at[pl.ds(t * 16, 16)], sem2))
        for g in gs:
            g.wait()
        for t in range(_CHUNK // 16):
            d16 = dest_c[pl.ds(j * _CHUNK + t * 16, 16)]
            ss.append(pltpu.async_copy(
                rowh_v.at[pl.ds(t * 16, 16)], out_hbm.at[d16], sem1))
            ss.append(pltpu.async_copy(
                rowv_v.at[pl.ds(t * 16, 16)], out_hbm.at[d16 + _N], sem2))
        for s in ss:
            s.wait()
        return 0

    lax.fori_loop(0, n_chunks, chunk_body, 0)


_sc_scatter = _mpmd._mpmd_map(
    [(_mesh, _sc_scatter_body)],
    jax.ShapeDtypeStruct((2 * _N, _D), jnp.float32),
    input_output_aliases={5: 0},
    scratch_types=[
        pltpu.VMEM((_CAP,), jnp.int32),
        pltpu.VMEM((_CAP,), jnp.int32),
        pltpu.VMEM((16,), jnp.int32),
        pltpu.VMEM((_CHUNK, _D), jnp.float32),
        pltpu.VMEM((_CHUNK, _D), jnp.float32),
        pltpu.SemaphoreType.DMA,
        pltpu.SemaphoreType.DMA,
    ],
    compiler_params=pltpu.CompilerParams(needs_layout_passes=False),
    cost_estimate=pl.CostEstimate(
        flops=1_000_000, bytes_accessed=33_000_000, transcendentals=0),
)


def kernel(x, idx, hidden, variance, W_ih, W_hh, b_ih, b_hh):
    idx = idx.astype(jnp.int32)
    h_old, var_old, dest, src, cnt = _sc_gather(idx, hidden, variance)
    stacked = _tc_copy(hidden, variance).reshape(2 * _N, _D)
    h_new, var_new = _tc_gru(
        x, h_old, var_old, W_ih.T, W_hh.T,
        b_ih.reshape(1, 3 * _D), b_hh.reshape(1, 3 * _D))
    out2 = _sc_scatter(h_new, var_new, dest, src, cnt, stacked)
    return out2.reshape(2, _N, _D)


# speedup vs baseline: 21.1385x; 1.0005x over previous
import functools

import jax
import jax.numpy as jnp
from jax import lax
from jax.experimental import pallas as pl
from jax.experimental.pallas import tpu as pltpu
from jax.experimental.pallas import tpu_sc as plsc
from jax._src.pallas import mpmd as _mpmd

_N = 100000
_D = 128
_B = 16384
_MOM = 0.9

_NC = 2
_NS = 16
_NW = _NC * _NS
_BPW = _B // _NW
_ROWS_PW = 3128
_MAP_VREGS = (_ROWS_PW + 15) // 16
_MAP_PAD = _MAP_VREGS * 16
_CAP = ((_ROWS_PW + 127) // 128 + 1) * 128
_CHUNK = 128

_mesh = plsc.VectorSubcoreMesh(
    core_axis_name="c", subcore_axis_name="s", num_cores=_NC, num_subcores=_NS
)


def _wid():
    return lax.axis_index("c") * _NS + lax.axis_index("s")


def _lane_gather(x, i):
    dnums = lax.GatherDimensionNumbers(
        offset_dims=(), collapsed_slice_dims=(0,), start_index_map=(0,))
    return lax.gather(x, i[:, None], dnums, (1,),
                      mode=lax.GatherScatterMode.PROMISE_IN_BOUNDS)


@functools.partial(
    pl.kernel,
    out_type=(
        jax.ShapeDtypeStruct((_B, _D), jnp.float32),
        jax.ShapeDtypeStruct((_B, _D), jnp.float32),
        jax.ShapeDtypeStruct((_NW, _CAP), jnp.int32),
        jax.ShapeDtypeStruct((_NW, _CAP), jnp.int32),
        jax.ShapeDtypeStruct((_NW, 16), jnp.int32),
    ),
    mesh=_mesh,
    scratch_types=[
        pltpu.VMEM((_B,), jnp.int32),
        pltpu.VMEM((_MAP_PAD,), jnp.int32),
        pltpu.VMEM((_CAP,), jnp.int32),
        pltpu.VMEM((_CAP,), jnp.int32),
        pltpu.VMEM((16,), jnp.int32),
        pltpu.VMEM((_CHUNK, _D), jnp.float32),
        pltpu.VMEM((_CHUNK, _D), jnp.float32),
        pltpu.VMEM((_CHUNK, _D), jnp.float32),
        pltpu.VMEM((_CHUNK, _D), jnp.float32),
        pltpu.SemaphoreType.DMA,
        pltpu.SemaphoreType.DMA,
        pltpu.SemaphoreType.DMA,
        pltpu.SemaphoreType.DMA,
        pltpu.SemaphoreType.DMA,
        pltpu.SemaphoreType.DMA,
        pltpu.SemaphoreType.DMA,
        pltpu.SemaphoreType.DMA,
    ],
    compiler_params=pltpu.CompilerParams(needs_layout_passes=False),
    cost_estimate=pl.CostEstimate(
        flops=2_000_000, bytes_accessed=40_000_000, transcendentals=0),
)
def _sc_gather(idx_hbm, hidden_hbm, variance_hbm,
               hold_hbm, varold_hbm, dest_hbm, src_hbm, cnt_hbm,
               idx_v, win_v, dest_c, src_c, cnt_v,
               hb0, hb1, vb0, vb1, semh0, semh1, semv0, semv1,
               wsemh0, wsemh1, wsemv0, wsemv1):
    wid = _wid()
    base = wid * _BPW
    lo = wid * _ROWS_PW
    hi = jnp.minimum(lo + _ROWS_PW, _N)

    pltpu.sync_copy(idx_hbm, idx_v)

    def _sl(ch):
        return idx_v.at[pl.ds(base + ch * _CHUNK, _CHUNK)]

    gh0 = pltpu.async_copy(hidden_hbm.at[_sl(0)], hb0, semh0)
    gh1 = pltpu.async_copy(hidden_hbm.at[_sl(1)], hb1, semh1)
    gv0 = pltpu.async_copy(variance_hbm.at[_sl(0)], vb0, semv0)
    gv1 = pltpu.async_copy(variance_hbm.at[_sl(1)], vb1, semv1)

    iot = lax.iota(jnp.int32, 16)

    def init_body(i, _):
        win_v[pl.ds(i * 16, 16)] = jnp.full((16,), -1, jnp.int32)
        return 0

    lax.fori_loop(0, _MAP_VREGS, init_body, 0)

    def scan_body(c, _):
        for u in range(2):
            cc = c * 2 + u
            chunk = idx_v[pl.ds(cc * 16, 16)]
            k2 = chunk * 16 + iot
            pos = cc * 16 + iot
            k2s, poss = plsc.sort_key_val(k2, pos)
            idxs = lax.shift_right_arithmetic(k2s, 4)
            nxt = _lane_gather(idxs, jnp.minimum(iot + 1, 15))
            bound = (idxs != nxt) | (iot == 15)
            inr = (idxs >= lo) & (idxs < hi)
            plsc.store_scatter(win_v, [idxs - lo], poss, mask=bound & inr)
        return 0

    lax.fori_loop(0, _B // 32, scan_body, 0)

    def compact_body(i, off):
        v = win_v[pl.ds(i * 16, 16)]
        m = v >= 0
        dvals = lo + i * 16 + iot
        plsc.store_compressed(dest_c.at[pl.ds(off, 16)], dvals, mask=m)
        plsc.store_compressed(src_c.at[pl.ds(off, 16)], v, mask=m)
        return off + jnp.sum(m.astype(jnp.int32))

    k_cnt = lax.fori_loop(0, _MAP_VREGS, compact_body, 0)

    @pl.when(k_cnt > 0)
    def _pad():
        zz = jnp.zeros((16,), jnp.int32)
        dpad = _lane_gather(dest_c[pl.ds(0, 16)], zz)
        spad = _lane_gather(src_c[pl.ds(0, 16)], zz)
        for t in range(_CHUNK // 16):
            dest_c[pl.ds(k_cnt + t * 16, 16)] = dpad
            src_c[pl.ds(k_cnt + t * 16, 16)] = spad

    cnt_v[...] = jnp.full((16,), k_cnt, jnp.int32)
    pltpu.sync_copy(dest_c, dest_hbm.at[wid])
    pltpu.sync_copy(src_c, src_hbm.at[wid])
    pltpu.sync_copy(cnt_v, cnt_hbm.at[wid])

    bufs = {"h": (hb0, hb1), "v": (vb0, vb1)}
    gsems = {"h": (semh0, semh1), "v": (semv0, semv1)}
    wsems = {"h": (wsemh0, wsemh1), "v": (wsemv0, wsemv1)}
    outs = {"h": hold_hbm, "v": varold_hbm}
    srcs = {"h": hidden_hbm, "v": variance_hbm}
    pend = {"h": [gh0, gh1], "v": [gv0, gv1]}
    wb = {"h": [None, None], "v": [None, None]}

    def _writeback(t, ch):
        return pltpu.async_copy(
            bufs[t][ch % 2], outs[t].at[pl.ds(base + ch * _CHUNK, _CHUNK)],
            wsems[t][ch % 2])

    for slot in range(2):
        for t in ("h", "v"):
            pend[t][slot].wait()
            wb[t][slot] = _writeback(t, slot)
    for slot in range(2):
        for t in ("h", "v"):
            wb[t][slot].wait()
            pend[t][slot] = pltpu.async_copy(
                srcs[t].at[_sl(slot + 2)], bufs[t][slot], gsems[t][slot])
    for slot in range(2):
        for t in ("h", "v"):
            pend[t][slot].wait()
            wb[t][slot] = _writeback(t, slot + 2)
    for slot in range(2):
        for t in ("h", "v"):
            wb[t][slot].wait()


_BM = 1024


def _gru_body(x_ref, h_ref, v_ref, wih_ref, whh_ref, bih_ref, bhh_ref,
              hn_ref, vn_ref):
    x = x_ref[...]
    h = h_ref[...]
    gi = jnp.dot(x, wih_ref[...], preferred_element_type=jnp.float32) + bih_ref[...]
    gh = jnp.dot(h, whh_ref[...], preferred_element_type=jnp.float32) + bhh_ref[...]
    r = jax.nn.sigmoid(gi[:, :_D] + gh[:, :_D])
    z = jax.nn.sigmoid(gi[:, _D:2 * _D] + gh[:, _D:2 * _D])
    n = jnp.tanh(gi[:, 2 * _D:] + r * gh[:, 2 * _D:])
    hn = (1.0 - z) * n + z * h
    hn_ref[...] = hn
    d = hn - h
    vn_ref[...] = _MOM * v_ref[...] + (1.0 - _MOM) * d * d


def _tc_gru(x, h_old, var_old, wih_t, whh_t, b_ih, b_hh):
    grid = (_B // _BM,)
    row_spec = pl.BlockSpec((_BM, _D), lambda i: (i, 0))
    full_w = pl.BlockSpec((_D, 3 * _D), lambda i: (0, 0))
    full_b = pl.BlockSpec((1, 3 * _D), lambda i: (0, 0))
    return pl.pallas_call(
        _gru_body,
        grid=grid,
        in_specs=[row_spec, row_spec, row_spec, full_w, full_w, full_b, full_b],
        out_specs=[row_spec, row_spec],
        out_shape=[
            jax.ShapeDtypeStruct((_B, _D), jnp.float32),
            jax.ShapeDtypeStruct((_B, _D), jnp.float32),
        ],
    )(x, h_old, var_old, wih_t, whh_t, b_ih, b_hh)


_CBLK = 5000


def _copy_body(h_ref, v_ref, out_ref):
    out_ref[0] = h_ref[...]
    out_ref[1] = v_ref[...]


def _tc_copy(hidden, variance):
    nb = _N // _CBLK
    blk = pl.BlockSpec((_CBLK, _D), lambda i: (i, 0))
    return pl.pallas_call(
        _copy_body,
        grid=(nb,),
        in_specs=[blk, blk],
        out_specs=pl.BlockSpec((2, _CBLK, _D), lambda i: (0, i, 0)),
        out_shape=jax.ShapeDtypeStruct((2, _N, _D), jnp.float32),
    )(hidden, variance)


def _sc_scatter_body(hnew_hbm, varnew_hbm, dest_hbm, src_hbm, cnt_hbm,
                     stacked_hbm, out_hbm,
                     dest_c, src_c, cnt_v, rowh_v, rowv_v, sem1, sem2):
    del stacked_hbm
    wid = _wid()

    c1 = pltpu.async_copy(dest_hbm.at[wid], dest_c, sem1)
    c2 = pltpu.async_copy(src_hbm.at[wid], src_c, sem2)
    c3 = pltpu.async_copy(cnt_hbm.at[wid], cnt_v, sem1)
    c1.wait()
    c2.wait()
    c3.wait()
    k_cnt = jnp.max(cnt_v[...], axis=0)
    n_chunks = (k_cnt + _CHUNK - 1) // _CHUNK

    def chunk_body(j, _):
        gs, ss = [], []
        for t in range(_CHUNK // 16):
            s16 = src_c[pl.ds(j * _CHUNK + t * 16, 16)]
            gs.append(pltpu.async_copy(
                hnew_hbm.at[s16], rowh_v.at[pl.ds(t * 16, 16)], sem1))
            gs.append(pltpu.async_copy(
                varnew_hbm.at[s16], rowv_v.at[pl.ds(t * 16, 16)], sem2))
        for g in gs:
            g.wait()
        for t in range(_CHUNK // 16):
            d16 = dest_c[pl.ds(j * _CHUNK + t * 16, 16)]
            ss.append(pltpu.async_copy(
                rowh_v.at[pl.ds(t * 16, 16)], out_hbm.at[d16], sem1))
            ss.append(pltpu.async_copy(
                rowv_v.at[pl.ds(t * 16, 16)], out_hbm.at[d16 + _N], sem2))
        for s in ss:
            s.wait()
        return 0

    lax.fori_loop(0, n_chunks, chunk_body, 0)


_sc_scatter = _mpmd._mpmd_map(
    [(_mesh, _sc_scatter_body)],
    jax.ShapeDtypeStruct((2 * _N, _D), jnp.float32),
    input_output_aliases={5: 0},
    scratch_types=[
        pltpu.VMEM((_CAP,), jnp.int32),
        pltpu.VMEM((_CAP,), jnp.int32),
        pltpu.VMEM((16,), jnp.int32),
        pltpu.VMEM((_CHUNK, _D), jnp.float32),
        pltpu.VMEM((_CHUNK, _D), jnp.float32),
        pltpu.SemaphoreType.DMA,
        pltpu.SemaphoreType.DMA,
    ],
    compiler_params=pltpu.CompilerParams(needs_layout_passes=False),
    cost_estimate=pl.CostEstimate(
        flops=1_000_000, bytes_accessed=33_000_000, transcendentals=0),
)


def kernel(x, idx, hidden, variance, W_ih, W_hh, b_ih, b_hh):
    idx = idx.astype(jnp.int32)
    h_old, var_old, dest, src, cnt = _sc_gather(idx, hidden, variance)
    stacked = _tc_copy(hidden, variance).reshape(2 * _N, _D)
    h_new, var_new = _tc_gru(
        x, h_old, var_old, W_ih.T, W_hh.T,
        b_ih.reshape(1, 3 * _D), b_hh.reshape(1, 3 * _D))
    out2 = _sc_scatter(h_new, var_new, dest, src, cnt, stacked)
    return out2.reshape(2, _N, _D)


# speedup vs baseline: 21.6789x; 1.0256x over previous
import functools

import jax
import jax.numpy as jnp
from jax import lax
from jax.experimental import pallas as pl
from jax.experimental.pallas import tpu as pltpu
from jax.experimental.pallas import tpu_sc as plsc
from jax._src.pallas import mpmd as _mpmd

_N = 100000
_D = 128
_B = 16384
_MOM = 0.9

_NC = 2
_NS = 16
_NW = _NC * _NS
_BPW = _B // _NW
_ROWS_PW = 3128
_MAP_VREGS = (_ROWS_PW + 15) // 16
_MAP_PAD = _MAP_VREGS * 16
_CAP = ((_ROWS_PW + 127) // 128 + 1) * 128
_CHUNK = 128

_mesh = plsc.VectorSubcoreMesh(
    core_axis_name="c", subcore_axis_name="s", num_cores=_NC, num_subcores=_NS
)


def _wid():
    return lax.axis_index("c") * _NS + lax.axis_index("s")


def _lane_gather(x, i):
    dnums = lax.GatherDimensionNumbers(
        offset_dims=(), collapsed_slice_dims=(0,), start_index_map=(0,))
    return lax.gather(x, i[:, None], dnums, (1,),
                      mode=lax.GatherScatterMode.PROMISE_IN_BOUNDS)


@functools.partial(
    pl.kernel,
    out_type=(
        jax.ShapeDtypeStruct((_B, _D), jnp.float32),
        jax.ShapeDtypeStruct((_B, _D), jnp.float32),
        jax.ShapeDtypeStruct((_NW, _CAP), jnp.int32),
        jax.ShapeDtypeStruct((_NW, _CAP), jnp.int32),
        jax.ShapeDtypeStruct((_NW, 16), jnp.int32),
    ),
    mesh=_mesh,
    scratch_types=[
        pltpu.VMEM((_B,), jnp.int32),
        pltpu.VMEM((_MAP_PAD,), jnp.int32),
        pltpu.VMEM((_CAP,), jnp.int32),
        pltpu.VMEM((_CAP,), jnp.int32),
        pltpu.VMEM((16,), jnp.int32),
        pltpu.VMEM((_CHUNK, _D), jnp.float32),
        pltpu.VMEM((_CHUNK, _D), jnp.float32),
        pltpu.VMEM((_CHUNK, _D), jnp.float32),
        pltpu.VMEM((_CHUNK, _D), jnp.float32),
        pltpu.SemaphoreType.DMA,
        pltpu.SemaphoreType.DMA,
        pltpu.SemaphoreType.DMA,
        pltpu.SemaphoreType.DMA,
        pltpu.SemaphoreType.DMA,
        pltpu.SemaphoreType.DMA,
        pltpu.SemaphoreType.DMA,
        pltpu.SemaphoreType.DMA,
    ],
    compiler_params=pltpu.CompilerParams(needs_layout_passes=False),
    cost_estimate=pl.CostEstimate(
        flops=2_000_000, bytes_accessed=40_000_000, transcendentals=0),
)
def _sc_gather(idx_hbm, hidden_hbm, variance_hbm,
               hold_hbm, varold_hbm, dest_hbm, src_hbm, cnt_hbm,
               idx_v, win_v, dest_c, src_c, cnt_v,
               hb0, hb1, vb0, vb1, semh0, semh1, semv0, semv1,
               wsemh0, wsemh1, wsemv0, wsemv1):
    wid = _wid()
    base = wid * _BPW
    lo = wid * _ROWS_PW
    hi = jnp.minimum(lo + _ROWS_PW, _N)

    pltpu.sync_copy(idx_hbm, idx_v)

    def _sl(ch):
        return idx_v.at[pl.ds(base + ch * _CHUNK, _CHUNK)]

    gh0 = pltpu.async_copy(hidden_hbm.at[_sl(0)], hb0, semh0)
    gh1 = pltpu.async_copy(hidden_hbm.at[_sl(1)], hb1, semh1)
    gv0 = pltpu.async_copy(variance_hbm.at[_sl(0)], vb0, semv0)
    gv1 = pltpu.async_copy(variance_hbm.at[_sl(1)], vb1, semv1)

    iot = lax.iota(jnp.int32, 16)

    def init_body(i, _):
        win_v[pl.ds(i * 16, 16)] = jnp.full((16,), -1, jnp.int32)
        return 0

    lax.fori_loop(0, _MAP_VREGS, init_body, 0)

    def scan_body(c, _):
        for u in range(4):
            cc = c * 4 + u
            chunk = idx_v[pl.ds(cc * 16, 16)]
            k2 = chunk * 16 + iot
            pos = cc * 16 + iot
            k2s, poss = plsc.sort_key_val(k2, pos)
            idxs = lax.shift_right_arithmetic(k2s, 4)
            nxt = _lane_gather(idxs, jnp.minimum(iot + 1, 15))
            bound = (idxs != nxt) | (iot == 15)
            inr = (idxs >= lo) & (idxs < hi)
            plsc.store_scatter(win_v, [idxs - lo], poss, mask=bound & inr)
        return 0

    lax.fori_loop(0, _B // 64, scan_body, 0)

    def compact_body(i, off):
        v = win_v[pl.ds(i * 16, 16)]
        m = v >= 0
        dvals = lo + i * 16 + iot
        plsc.store_compressed(dest_c.at[pl.ds(off, 16)], dvals, mask=m)
        plsc.store_compressed(src_c.at[pl.ds(off, 16)], v, mask=m)
        return off + jnp.sum(m.astype(jnp.int32))

    k_cnt = lax.fori_loop(0, _MAP_VREGS, compact_body, 0)

    @pl.when(k_cnt > 0)
    def _pad():
        zz = jnp.zeros((16,), jnp.int32)
        dpad = _lane_gather(dest_c[pl.ds(0, 16)], zz)
        spad = _lane_gather(src_c[pl.ds(0, 16)], zz)
        for t in range(_CHUNK // 16):
            dest_c[pl.ds(k_cnt + t * 16, 16)] = dpad
            src_c[pl.ds(k_cnt + t * 16, 16)] = spad

    cnt_v[...] = jnp.full((16,), k_cnt, jnp.int32)
    pltpu.sync_copy(dest_c, dest_hbm.at[wid])
    pltpu.sync_copy(src_c, src_hbm.at[wid])
    pltpu.sync_copy(cnt_v, cnt_hbm.at[wid])

    bufs = {"h": (hb0, hb1), "v": (vb0, vb1)}
    gsems = {"h": (semh0, semh1), "v": (semv0, semv1)}
    wsems = {"h": (wsemh0, wsemh1), "v": (wsemv0, wsemv1)}
    outs = {"h": hold_hbm, "v": varold_hbm}
    srcs = {"h": hidden_hbm, "v": variance_hbm}
    pend = {"h": [gh0, gh1], "v": [gv0, gv1]}
    wb = {"h": [None, None], "v": [None, None]}

    def _writeback(t, ch):
        return pltpu.async_copy(
            bufs[t][ch % 2], outs[t].at[pl.ds(base + ch * _CHUNK, _CHUNK)],
            wsems[t][ch % 2])

    for slot in range(2):
        for t in ("h", "v"):
            pend[t][slot].wait()
            wb[t][slot] = _writeback(t, slot)
    for slot in range(2):
        for t in ("h", "v"):
            wb[t][slot].wait()
            pend[t][slot] = pltpu.async_copy(
                srcs[t].at[_sl(slot + 2)], bufs[t][slot], gsems[t][slot])
    for slot in range(2):
        for t in ("h", "v"):
            pend[t][slot].wait()
            wb[t][slot] = _writeback(t, slot + 2)
    for slot in range(2):
        for t in ("h", "v"):
            wb[t][slot].wait()


_BM = 2048


def _gru_body(x_ref, h_ref, v_ref, wih_ref, whh_ref, bih_ref, bhh_ref,
              hn_ref, vn_ref):
    x = x_ref[...]
    h = h_ref[...]
    gi = jnp.dot(x, wih_ref[...], preferred_element_type=jnp.float32) + bih_ref[...]
    gh = jnp.dot(h, whh_ref[...], preferred_element_type=jnp.float32) + bhh_ref[...]
    r = jax.nn.sigmoid(gi[:, :_D] + gh[:, :_D])
    z = jax.nn.sigmoid(gi[:, _D:2 * _D] + gh[:, _D:2 * _D])
    n = jnp.tanh(gi[:, 2 * _D:] + r * gh[:, 2 * _D:])
    hn = (1.0 - z) * n + z * h
    hn_ref[...] = hn
    d = hn - h
    vn_ref[...] = _MOM * v_ref[...] + (1.0 - _MOM) * d * d


def _tc_gru(x, h_old, var_old, wih_t, whh_t, b_ih, b_hh):
    grid = (_B // _BM,)
    row_spec = pl.BlockSpec((_BM, _D), lambda i: (i, 0))
    full_w = pl.BlockSpec((_D, 3 * _D), lambda i: (0, 0))
    full_b = pl.BlockSpec((1, 3 * _D), lambda i: (0, 0))
    return pl.pallas_call(
        _gru_body,
        grid=grid,
        in_specs=[row_spec, row_spec, row_spec, full_w, full_w, full_b, full_b],
        out_specs=[row_spec, row_spec],
        out_shape=[
            jax.ShapeDtypeStruct((_B, _D), jnp.float32),
            jax.ShapeDtypeStruct((_B, _D), jnp.float32),
        ],
    )(x, h_old, var_old, wih_t, whh_t, b_ih, b_hh)


_CBLK = 5000


def _copy_body(h_ref, v_ref, out_ref):
    out_ref[0] = h_ref[...]
    out_ref[1] = v_ref[...]


def _tc_copy(hidden, variance):
    nb = _N // _CBLK
    blk = pl.BlockSpec((_CBLK, _D), lambda i: (i, 0))
    return pl.pallas_call(
        _copy_body,
        grid=(nb,),
        in_specs=[blk, blk],
        out_specs=pl.BlockSpec((2, _CBLK, _D), lambda i: (0, i, 0)),
        out_shape=jax.ShapeDtypeStruct((2, _N, _D), jnp.float32),
    )(hidden, variance)


def _sc_scatter_body(hnew_hbm, varnew_hbm, dest_hbm, src_hbm, cnt_hbm,
                     stacked_hbm, out_hbm,
                     dest_c, src_c, cnt_v, rowh_v, rowv_v, sem1, sem2):
    del stacked_hbm
    wid = _wid()

    c1 = pltpu.async_copy(dest_hbm.at[wid], dest_c, sem1)
    c2 = pltpu.async_copy(src_hbm.at[wid], src_c, sem2)
    c3 = pltpu.async_copy(cnt_hbm.at[wid], cnt_v, sem1)
    c1.wait()
    c2.wait()
    c3.wait()
    k_cnt = jnp.max(cnt_v[...], axis=0)
    n_chunks = (k_cnt + _CHUNK - 1) // _CHUNK

    def chunk_body(j, _):
        gs, ss = [], []
        for t in range(_CHUNK // 16):
            s16 = src_c[pl.ds(j * _CHUNK + t * 16, 16)]
            gs.append(pltpu.async_copy(
                hnew_hbm.at[s16], rowh_v.at[pl.ds(t * 16, 16)], sem1))
            gs.append(pltpu.async_copy(
                varnew_hbm.at[s16], rowv_v.at[pl.ds(t * 16, 16)], sem2))
        for g in gs:
            g.wait()
        for t in range(_CHUNK // 16):
            d16 = dest_c[pl.ds(j * _CHUNK + t * 16, 16)]
            ss.append(pltpu.async_copy(
                rowh_v.at[pl.ds(t * 16, 16)], out_hbm.at[d16], sem1))
            ss.append(pltpu.async_copy(
                rowv_v.at[pl.ds(t * 16, 16)], out_hbm.at[d16 + _N], sem2))
        for s in ss:
            s.wait()
        return 0

    lax.fori_loop(0, n_chunks, chunk_body, 0)


_sc_scatter = _mpmd._mpmd_map(
    [(_mesh, _sc_scatter_body)],
    jax.ShapeDtypeStruct((2 * _N, _D), jnp.float32),
    input_output_aliases={5: 0},
    scratch_types=[
        pltpu.VMEM((_CAP,), jnp.int32),
        pltpu.VMEM((_CAP,), jnp.int32),
        pltpu.VMEM((16,), jnp.int32),
        pltpu.VMEM((_CHUNK, _D), jnp.float32),
        pltpu.VMEM((_CHUNK, _D), jnp.float32),
        pltpu.SemaphoreType.DMA,
        pltpu.SemaphoreType.DMA,
    ],
    compiler_params=pltpu.CompilerParams(needs_layout_passes=False),
    cost_estimate=pl.CostEstimate(
        flops=1_000_000, bytes_accessed=33_000_000, transcendentals=0),
)


def kernel(x, idx, hidden, variance, W_ih, W_hh, b_ih, b_hh):
    idx = idx.astype(jnp.int32)
    h_old, var_old, dest, src, cnt = _sc_gather(idx, hidden, variance)
    stacked = _tc_copy(hidden, variance).reshape(2 * _N, _D)
    h_new, var_new = _tc_gru(
        x, h_old, var_old, W_ih.T, W_hh.T,
        b_ih.reshape(1, 3 * _D), b_hh.reshape(1, 3 * _D))
    out2 = _sc_scatter(h_new, var_new, dest, src, cnt, stacked)
    return out2.reshape(2, _N, _D)
